# Initial kernel scaffold; baseline (speedup 1.0000x reference)
import jax
import jax.numpy as jnp
from jax.experimental import pallas as pl


def kernel(z_t, h_t, mem_bank, Wq, bq, Wc, bc, Ws, bs, Wm, bm, Wg1, bg1, Wg2, bg2):
    raise NotImplementedError("write your pallas kernel here")


# speedup vs baseline: 1.9982x; 1.9982x over previous
import functools

import jax
import jax.numpy as jnp
from jax import lax
from jax.experimental import pallas as pl
from jax.experimental.pallas import tpu as pltpu
from jax.experimental.pallas import tpu_sc as plsc

N_ROWS = 1_000_000
D = 64
K_MEM = 8
N_DREAMS = 4

ROWS_PER_BLK = 32768
S_SUB = ROWS_PER_BLK // 128
KDIM = 128 * D
NBLK = (N_ROWS + ROWS_PER_BLK - 1) // ROWS_PER_BLK
NCAND = 256


def _sims_topk_body(blk_ref, z2_ref, vals_ref, idx_ref):
    i = pl.program_id(0)
    blk = blk_ref[...]
    dots = jnp.sum(blk * z2_ref[...], axis=1)
    sq = jnp.sum(blk * blk, axis=1)
    sims = dots * lax.rsqrt(jnp.maximum(sq, 1e-24))
    gidx = i * ROWS_PER_BLK + lax.broadcasted_iota(
        jnp.int32, (ROWS_PER_BLK,), 0)
    sims = jnp.where(gidx < N_ROWS, sims, -jnp.inf)
    lane = lax.broadcasted_iota(jnp.int32, (1, 128), 1)
    bv = jnp.full((1, 128), -jnp.inf, jnp.float32)
    bi = jnp.zeros((1, 128), jnp.int32)
    for t in range(K_MEM):
        m = jnp.max(sims)
        pos = jnp.min(jnp.where(sims == m, gidx, jnp.int32(2**31 - 1)))
        bv = jnp.where(lane == t, m, bv)
        bi = jnp.where(lane == t, pos, bi)
        sims = jnp.where(gidx == pos, -jnp.inf, sims)
    vals_ref[...] = bv[None, :, :]
    idx_ref[...] = bi[None, :, :]


def _sc_merge_body(cv_hbm, ci_hbm, out_hbm, cv_v, ci_v, idx_v):
    c = lax.axis_index("c")
    s = lax.axis_index("s")

    @pl.when(jnp.logical_and(c == 0, s == 0))
    def _():
        pltpu.sync_copy(cv_hbm, cv_v)
        pltpu.sync_copy(ci_hbm, ci_v)
        rv = cv_v[pl.ds(0, 16)]
        ri = ci_v[pl.ds(0, 16)]
        rv, ri = plsc.sort_key_val(rv, ri, descending=True)
        for j in range(1, NCAND // 16):
            av = cv_v[pl.ds(j * 16, 16)]
            ai = ci_v[pl.ds(j * 16, 16)]
            av, ai = plsc.sort_key_val(av, ai, descending=True)
            bvv = lax.rev(rv, (0,))
            bii = lax.rev(ri, (0,))
            take = av >= bvv
            rv = jnp.where(take, av, bvv)
            ri = jnp.where(take, ai, bii)
            rv, ri = plsc.sort_key_val(rv, ri, descending=True)
        idx_v[...] = ri
        pltpu.sync_copy(idx_v, out_hbm)


def _gather_mlp_body(sref, mem_ref, z_ref, h_ref, nz_ref, wq_ref, bq_ref,
                     wc_ref, bc_ref, ws_ref, bs_ref, wm_ref, bm_ref,
                     wg1_ref, bg1_ref, wg2_ref, bg2_ref, out_ref, protos_s):
    i = pl.program_id(0)
    sub = sref[i] % 8
    riota = lax.broadcasted_iota(jnp.int32, (8, 1), 0)
    row = jnp.sum(jnp.where(riota == sub, mem_ref[...], 0.0), axis=0,
                  keepdims=True)
    protos_s[pl.ds(i, 1), :] = row

    @pl.when(i == K_MEM - 1)
    def _():
        dn = (((1,), (0,)), ((), ()))
        z = z_ref[...]
        h = h_ref[...]
        d_raw = 0.7 * nz_ref[...] + 0.3 * z
        d_norm = jnp.sqrt(jnp.sum(d_raw * d_raw, axis=1, keepdims=True))
        dreams = d_raw / jnp.maximum(d_norm, 1e-12)
        C = jnp.concatenate([protos_s[...], dreams], axis=0)
        ctx = jnp.concatenate([z, h], axis=1)
        q_vec = jnp.tanh(
            lax.dot_general(ctx, wq_ref[...], dn,
                            preferred_element_type=jnp.float32) + bq_ref[...])
        C_proj = lax.dot_general(C, wc_ref[...], dn,
                                 preferred_element_type=jnp.float32
                                 ) + bc_ref[...]
        logits = (jnp.sum(C_proj * q_vec * ws_ref[...], axis=1, keepdims=True)
                  + bs_ref[...])
        e = jnp.exp(logits - jnp.max(logits))
        attn = e / jnp.sum(e)
        s_raw = jnp.sum(attn * C, axis=0, keepdims=True)
        s_mixed = jnp.tanh(
            lax.dot_general(s_raw, wm_ref[...], dn,
                            preferred_element_type=jnp.float32) + bm_ref[...])
        u = jnp.tanh(
            lax.dot_general(ctx, wg1_ref[...], dn,
                            preferred_element_type=jnp.float32) + bg1_ref[...])
        g = jax.nn.sigmoid(
            jnp.sum(u * wg2_ref[...], axis=1, keepdims=True) + bg2_ref[...])
        out_ref[...] = g * s_mixed


def _stage1(mem_bank, z2):
    return pl.pallas_call(
        _sims_topk_body,
        grid=(NBLK,),
        in_specs=[
            pl.BlockSpec((ROWS_PER_BLK, D), lambda i: (i, 0)),
            pl.BlockSpec((1, D), lambda i: (0, 0)),
        ],
        out_specs=[
            pl.BlockSpec((1, 1, 128), lambda i: (i, 0, 0)),
            pl.BlockSpec((1, 1, 128), lambda i: (i, 0, 0)),
        ],
        out_shape=[
            jax.ShapeDtypeStruct((NBLK, 1, 128), jnp.float32),
            jax.ShapeDtypeStruct((NBLK, 1, 128), jnp.int32),
        ],
    )(mem_bank, z2)


def _stage2_sc(cand_v, cand_i):
    mesh = plsc.VectorSubcoreMesh(core_axis_name="c", subcore_axis_name="s")
    return pl.kernel(
        _sc_merge_body,
        mesh=mesh,
        compiler_params=pltpu.CompilerParams(needs_layout_passes=False),
        out_type=jax.ShapeDtypeStruct((16,), jnp.int32),
        scratch_types=[
            pltpu.VMEM((NCAND,), jnp.float32),
            pltpu.VMEM((NCAND,), jnp.int32),
            pltpu.VMEM((16,), jnp.int32),
        ],
    )(cand_v, cand_i)


def _stage3(sidx, mem_bank, z2, h2, noise, WqT, bq2, WcT, bc2, Ws2, bs2,
            WmT, bm2, Wg1T, bg12, Wg22, bg22):
    full = lambda shape: pl.BlockSpec(shape, lambda i, s: (0,) * len(shape))
    args = (z2, h2, noise, WqT, bq2, WcT, bc2, Ws2, bs2, WmT, bm2,
            Wg1T, bg12, Wg22, bg22)
    grid_spec = pltpu.PrefetchScalarGridSpec(
        num_scalar_prefetch=1,
        grid=(K_MEM,),
        in_specs=[pl.BlockSpec((8, D), lambda i, s: (s[i] // 8, 0))]
        + [full(a.shape) for a in args],
        out_specs=full((1, D)),
        scratch_shapes=[pltpu.VMEM((K_MEM, D), jnp.float32)],
    )
    return pl.pallas_call(
        _gather_mlp_body,
        grid_spec=grid_spec,
        out_shape=jax.ShapeDtypeStruct((1, D), jnp.float32),
    )(sidx, mem_bank, *args)


def kernel(z_t, h_t, mem_bank, Wq, bq, Wc, bc, Ws, bs, Wm, bm, Wg1, bg1,
           Wg2, bg2):
    f32 = jnp.float32
    vals, idx = _stage1(mem_bank, z_t.reshape(1, D))

    pad = NCAND - NBLK * K_MEM
    cand_v = jnp.concatenate(
        [vals[:, 0, :K_MEM].reshape(-1), jnp.full((pad,), -jnp.inf, f32)])
    cand_i = jnp.concatenate(
        [idx[:, 0, :K_MEM].reshape(-1), jnp.zeros((pad,), jnp.int32)])

    top_idx = _stage2_sc(cand_v, cand_i)
    sidx = top_idx[:K_MEM]

    dkey = jax.random.key(42)
    noise = jnp.stack([
        jax.random.normal(jax.random.fold_in(dkey, i), (D,), dtype=f32)
        for i in range(N_DREAMS)
    ])

    out = _stage3(
        sidx, mem_bank,
        z_t.reshape(1, D), h_t.reshape(1, D), noise,
        Wq.T, bq.reshape(1, D),
        Wc.T, bc.reshape(1, D),
        Ws.reshape(1, D), bs.reshape(1, 1),
        Wm.T, bm.reshape(1, D),
        Wg1.T, bg1.reshape(1, D),
        Wg2.reshape(1, D), bg2.reshape(1, 1),
    )
    return out.reshape(D)


# speedup vs baseline: 2.9497x; 1.4761x over previous
import functools

import jax
import jax.numpy as jnp
from jax import lax
from jax.experimental import pallas as pl
from jax.experimental.pallas import tpu as pltpu
from jax.experimental.pallas import tpu_sc as plsc

N_ROWS = 1_000_000
D = 64
K_MEM = 8
N_DREAMS = 4

ROWS_PER_BLK = 32768
NBLK = (N_ROWS + ROWS_PER_BLK - 1) // ROWS_PER_BLK
NPAD = NBLK * ROWS_PER_BLK
NWORKER = 32
CHUNK = NPAD // NWORKER
NCAND = NWORKER * 16


def _sims_body(blk_ref, z2_ref, sims_ref):
    i = pl.program_id(0)
    blk = blk_ref[...]
    ddn = (((1,), (1,)), ((), ()))
    dots = lax.dot_general(z2_ref[...], blk, ddn,
                           preferred_element_type=jnp.float32)
    sq = lax.dot_general(jnp.ones((1, D), jnp.float32), blk * blk, ddn,
                         preferred_element_type=jnp.float32)
    sims = dots * lax.rsqrt(jnp.maximum(sq, 1e-24))
    gidx = i * ROWS_PER_BLK + lax.broadcasted_iota(
        jnp.int32, (1, ROWS_PER_BLK), 1)
    sims = jnp.where(gidx < N_ROWS, sims, -jnp.inf)
    sims_ref[...] = sims[None, :, :]


def _merge16(rv, ri, av, ai):
    av, ai = plsc.sort_key_val(av, ai, descending=True)
    bv = lax.rev(rv, (0,))
    bi = lax.rev(ri, (0,))
    take = av >= bv
    rv = jnp.where(take, av, bv)
    ri = jnp.where(take, ai, bi)
    rv, ri = plsc.sort_key_val(rv, ri, descending=True)
    return rv, ri


def _sc_local_topk_body(sims_hbm, vals_hbm, idx_hbm, chunk_v, v16_v, i16_v):
    c = lax.axis_index("c")
    s = lax.axis_index("s")
    wid = s * 2 + c
    base = wid * CHUNK
    pltpu.sync_copy(sims_hbm.at[pl.ds(base, CHUNK)], chunk_v)
    iota16 = jax.lax.iota(jnp.int32, 16)
    rv = chunk_v[pl.ds(0, 16)]
    ri = base + iota16
    rv, ri = plsc.sort_key_val(rv, ri, descending=True)

    def body(j, carry):
        rv, ri = carry
        off = pl.multiple_of(j * 16, 16)
        av = chunk_v[pl.ds(off, 16)]
        ai = base + j * 16 + iota16
        return _merge16(rv, ri, av, ai)

    rv, ri = lax.fori_loop(1, CHUNK // 16, body, (rv, ri))
    v16_v[...] = rv
    i16_v[...] = ri
    pltpu.sync_copy(v16_v, vals_hbm.at[wid])
    pltpu.sync_copy(i16_v, idx_hbm.at[wid])


def _sc_merge_body(cv_hbm, ci_hbm, out_hbm, cv_v, ci_v, idx_v):
    c = lax.axis_index("c")
    s = lax.axis_index("s")

    @pl.when(jnp.logical_and(c == 0, s == 0))
    def _():
        pltpu.sync_copy(cv_hbm, cv_v)
        pltpu.sync_copy(ci_hbm, ci_v)
        rv = cv_v[pl.ds(0, 16)]
        ri = ci_v[pl.ds(0, 16)]
        rv, ri = plsc.sort_key_val(rv, ri, descending=True)
        for j in range(1, NCAND // 16):
            av = cv_v[pl.ds(j * 16, 16)]
            ai = ci_v[pl.ds(j * 16, 16)]
            rv, ri = _merge16(rv, ri, av, ai)
        idx_v[...] = ri
        pltpu.sync_copy(idx_v, out_hbm)


def _gather_mlp_body(sref, mem_ref, z_ref, h_ref, nz_ref, wq_ref, bq_ref,
                     wc_ref, bc_ref, ws_ref, bs_ref, wm_ref, bm_ref,
                     wg1_ref, bg1_ref, wg2_ref, bg2_ref, out_ref, protos_s):
    i = pl.program_id(0)
    sub = sref[i] % 8
    riota = lax.broadcasted_iota(jnp.int32, (8, 1), 0)
    row = jnp.sum(jnp.where(riota == sub, mem_ref[...], 0.0), axis=0,
                  keepdims=True)
    protos_s[pl.ds(i, 1), :] = row

    @pl.when(i == K_MEM - 1)
    def _():
        dn = (((1,), (0,)), ((), ()))
        z = z_ref[...]
        h = h_ref[...]
        d_raw = 0.7 * nz_ref[...] + 0.3 * z
        d_norm = jnp.sqrt(jnp.sum(d_raw * d_raw, axis=1, keepdims=True))
        dreams = d_raw / jnp.maximum(d_norm, 1e-12)
        C = jnp.concatenate([protos_s[...], dreams], axis=0)
        ctx = jnp.concatenate([z, h], axis=1)
        q_vec = jnp.tanh(
            lax.dot_general(ctx, wq_ref[...], dn,
                            preferred_element_type=jnp.float32) + bq_ref[...])
        C_proj = lax.dot_general(C, wc_ref[...], dn,
                                 preferred_element_type=jnp.float32
                                 ) + bc_ref[...]
        logits = (jnp.sum(C_proj * q_vec * ws_ref[...], axis=1, keepdims=True)
                  + bs_ref[...])
        e = jnp.exp(logits - jnp.max(logits))
        attn = e / jnp.sum(e)
        s_raw = jnp.sum(attn * C, axis=0, keepdims=True)
        s_mixed = jnp.tanh(
            lax.dot_general(s_raw, wm_ref[...], dn,
                            preferred_element_type=jnp.float32) + bm_ref[...])
        u = jnp.tanh(
            lax.dot_general(ctx, wg1_ref[...], dn,
                            preferred_element_type=jnp.float32) + bg1_ref[...])
        g = jax.nn.sigmoid(
            jnp.sum(u * wg2_ref[...], axis=1, keepdims=True) + bg2_ref[...])
        out_ref[...] = g * s_mixed


def _stage1(mem_bank, z2):
    return pl.pallas_call(
        _sims_body,
        grid=(NBLK,),
        in_specs=[
            pl.BlockSpec((ROWS_PER_BLK, D), lambda i: (i, 0)),
            pl.BlockSpec((1, D), lambda i: (0, 0)),
        ],
        out_specs=pl.BlockSpec((1, 1, ROWS_PER_BLK), lambda i: (i, 0, 0)),
        out_shape=jax.ShapeDtypeStruct((NBLK, 1, ROWS_PER_BLK), jnp.float32),
    )(mem_bank, z2)


def _sc_mesh():
    return plsc.VectorSubcoreMesh(core_axis_name="c", subcore_axis_name="s")


def _stage2a_sc(sims_flat):
    return pl.kernel(
        _sc_local_topk_body,
        mesh=_sc_mesh(),
        compiler_params=pltpu.CompilerParams(needs_layout_passes=False),
        out_type=[
            jax.ShapeDtypeStruct((NWORKER, 16), jnp.float32),
            jax.ShapeDtypeStruct((NWORKER, 16), jnp.int32),
        ],
        scratch_types=[
            pltpu.VMEM((CHUNK,), jnp.float32),
            pltpu.VMEM((16,), jnp.float32),
            pltpu.VMEM((16,), jnp.int32),
        ],
    )(sims_flat)


def _stage2b_sc(cand_v, cand_i):
    return pl.kernel(
        _sc_merge_body,
        mesh=_sc_mesh(),
        compiler_params=pltpu.CompilerParams(needs_layout_passes=False),
        out_type=jax.ShapeDtypeStruct((16,), jnp.int32),
        scratch_types=[
            pltpu.VMEM((NCAND,), jnp.float32),
            pltpu.VMEM((NCAND,), jnp.int32),
            pltpu.VMEM((16,), jnp.int32),
        ],
    )(cand_v, cand_i)


def _stage3(sidx, mem_bank, z2, h2, noise, WqT, bq2, WcT, bc2, Ws2, bs2,
            WmT, bm2, Wg1T, bg12, Wg22, bg22):
    full = lambda shape: pl.BlockSpec(shape, lambda i, s: (0,) * len(shape))
    args = (z2, h2, noise, WqT, bq2, WcT, bc2, Ws2, bs2, WmT, bm2,
            Wg1T, bg12, Wg22, bg22)
    grid_spec = pltpu.PrefetchScalarGridSpec(
        num_scalar_prefetch=1,
        grid=(K_MEM,),
        in_specs=[pl.BlockSpec((8, D), lambda i, s: (s[i] // 8, 0))]
        + [full(a.shape) for a in args],
        out_specs=full((1, D)),
        scratch_shapes=[pltpu.VMEM((K_MEM, D), jnp.float32)],
    )
    return pl.pallas_call(
        _gather_mlp_body,
        grid_spec=grid_spec,
        out_shape=jax.ShapeDtypeStruct((1, D), jnp.float32),
    )(sidx, mem_bank, *args)


def kernel(z_t, h_t, mem_bank, Wq, bq, Wc, bc, Ws, bs, Wm, bm, Wg1, bg1,
           Wg2, bg2):
    f32 = jnp.float32
    sims3 = _stage1(mem_bank, z_t.reshape(1, D))
    sims_flat = sims3.reshape(NPAD)

    vals, idx = _stage2a_sc(sims_flat)
    top_idx = _stage2b_sc(vals.reshape(NCAND), idx.reshape(NCAND))
    sidx = top_idx[:K_MEM]

    dkey = jax.random.key(42)
    noise = jnp.stack([
        jax.random.normal(jax.random.fold_in(dkey, i), (D,), dtype=f32)
        for i in range(N_DREAMS)
    ])

    out = _stage3(
        sidx, mem_bank,
        z_t.reshape(1, D), h_t.reshape(1, D), noise,
        Wq.T, bq.reshape(1, D),
        Wc.T, bc.reshape(1, D),
        Ws.reshape(1, D), bs.reshape(1, 1),
        Wm.T, bm.reshape(1, D),
        Wg1.T, bg1.reshape(1, D),
        Wg2.reshape(1, D), bg2.reshape(1, 1),
    )
    return out.reshape(D)
